# baseline (speedup 1.0000x reference)
import functools

import jax
import jax.numpy as jnp
from jax import lax
from jax.experimental import pallas as pl
from jax.experimental.pallas import tpu as pltpu
from jax.experimental.pallas import tpu_sc as plsc

N_ROWS = 320000
D = 128
N_WORKERS = 32
ROWS_PER_W = N_ROWS // N_WORKERS
CHUNK = 200
N_CHUNKS = ROWS_PER_W // CHUNK
NV = D // 16


def _sc_body(feat_hbm, lbl_hbm, edge_out, meta_out,
             lbl_v, fbuf, ebuf, mbuf, sem0, sem1):
    c = lax.axis_index("c")
    s = lax.axis_index("s")
    wid = s * 2 + c
    row0 = wid * ROWS_PER_W

    pltpu.sync_copy(lbl_hbm.at[pl.ds(row0, ROWS_PER_W)],
                    lbl_v.at[pl.ds(0, ROWS_PER_W)])

    sems = (sem0, sem1)

    def start(ci, b):
        pltpu.async_copy(feat_hbm.at[pl.ds(row0 + ci * CHUNK, CHUNK)],
                         fbuf.at[b], sems[b])

    def wait(ci, b):
        pltpu.make_async_copy(feat_hbm.at[pl.ds(row0 + ci * CHUNK, CHUNK)],
                              fbuf.at[b], sems[b]).wait()

    start(0, 0)
    start(1, 1)

    zero_v = jnp.zeros((16,), jnp.float32)
    init = dict(
        prev=jnp.int32(-1),
        cnt=jnp.float32(0.0),
        nseg=jnp.int32(0),
        interior=zero_v,
        flbl=jnp.float32(0.0),
        fcnt=jnp.float32(0.0),
        acc=[zero_v] * NV,
        sq=[zero_v] * NV,
        facc=[zero_v] * NV,
    )

    def row_step(b, gr0, r, st):
        gr = gr0 + r
        lbl = lbl_v[pl.ds(gr, 16)][0]
        boundary = jnp.logical_and(gr > 0, lbl != st["prev"])
        is_first = st["nseg"] == 0
        bf = jnp.logical_and(boundary, is_first)
        bi = jnp.logical_and(boundary, jnp.logical_not(is_first))

        acc = st["acc"]
        n2v = acc[0] * acc[0]
        for j in range(1, NV):
            n2v = n2v + acc[j] * acc[j]
        cntv = zero_v + st["cnt"]
        wv = jnp.where(bi, 1.0 / jnp.maximum(cntv, 1.0), zero_v)
        interior = st["interior"] + n2v * wv

        flbl = jnp.where(bf, st["prev"].astype(jnp.float32), st["flbl"])
        fcnt = jnp.where(bf, st["cnt"], st["fcnt"])
        facc = [jnp.where(bf, acc[j], st["facc"][j]) for j in range(NV)]
        nseg = jnp.where(boundary, st["nseg"] + 1, st["nseg"])
        cnt = jnp.where(boundary, jnp.float32(0.0), st["cnt"]) + 1.0

        sq = st["sq"]
        new_acc = []
        new_sq = []
        for j in range(NV):
            v = fbuf[b, r, pl.ds(j * 16, 16)]
            a = jnp.where(boundary, zero_v, acc[j])
            new_acc.append(a + v)
            new_sq.append(sq[j] + v * v)
        return dict(prev=lbl, cnt=cnt, nseg=nseg, interior=interior,
                    flbl=flbl, fcnt=fcnt, acc=new_acc, sq=new_sq, facc=facc)

    def row_quad(b, gr0, r, st):
        for u in range(4):
            st = row_step(b, gr0, 4 * r + u, st)
        return st

    def chunk_pair(k, st):
        for b in range(2):
            ci = 2 * k + b
            wait(ci, b)
            st = lax.fori_loop(
                0, CHUNK // 4,
                functools.partial(row_quad, b, ci * CHUNK),
                st)

            @pl.when(ci + 2 < N_CHUNKS)
            def _():
                start(ci + 2, b)
        return st

    st = lax.fori_loop(0, N_CHUNKS // 2, chunk_pair, init)

    nseg = st["nseg"]
    last_lbl = st["prev"].astype(jnp.float32)
    flbl = jnp.where(nseg == 0, last_lbl, st["flbl"])
    fcnt = st["fcnt"]
    sqv = st["sq"][0]
    for j in range(1, NV):
        sqv = sqv + st["sq"][j]
    for j in range(NV):
        ebuf[0, pl.ds(j * 16, 16)] = st["facc"][j]
        ebuf[1, pl.ds(j * 16, 16)] = st["acc"][j]

    ii = lax.iota(jnp.int32, 16)
    zf = jnp.zeros((16,), jnp.float32)
    mfirst = jnp.where(ii == 0, flbl, zf) + jnp.where(ii == 1, fcnt, zf)
    mlast = (jnp.where(ii == 0, last_lbl, zf)
             + jnp.where(ii == 1, st["cnt"], zf))
    mbuf[0, pl.ds(0, 16)] = mfirst
    mbuf[1, pl.ds(0, 16)] = mlast
    mbuf[2, pl.ds(0, 16)] = sqv
    mbuf[3, pl.ds(0, 16)] = st["interior"]

    pltpu.sync_copy(ebuf, edge_out.at[pl.ds(2 * wid, 2)])
    pltpu.sync_copy(mbuf.at[pl.ds(0, 2)], meta_out.at[pl.ds(2 * wid, 2)])
    pltpu.sync_copy(mbuf.at[pl.ds(2, 1)], meta_out.at[pl.ds(2 * N_WORKERS + wid, 1)])
    pltpu.sync_copy(mbuf.at[pl.ds(3, 1)],
                    meta_out.at[pl.ds(3 * N_WORKERS + wid, 1)])


_sc_pass = pl.kernel(
    _sc_body,
    out_type=[
        jax.ShapeDtypeStruct((2 * N_WORKERS, D), jnp.float32),
        jax.ShapeDtypeStruct((4 * N_WORKERS, 16), jnp.float32),
    ],
    mesh=plsc.VectorSubcoreMesh(core_axis_name="c", subcore_axis_name="s"),
    scratch_types=[
        pltpu.VMEM((ROWS_PER_W + 16,), jnp.int32),
        pltpu.VMEM((2, CHUNK, D), jnp.float32),
        pltpu.VMEM((2, D), jnp.float32),
        pltpu.VMEM((4, 16), jnp.float32),
        pltpu.SemaphoreType.DMA,
        pltpu.SemaphoreType.DMA,
    ],
)


def _tc_combine_body(edge_ref, meta_ref, out_ref):
    ne = 2 * N_WORKERS
    e = edge_ref[...]
    m = meta_ref[...]
    lbl = m[:ne, 0:1]
    cnt = m[:ne, 1:2]
    sx = jnp.sum(m[ne:ne + N_WORKERS, :])
    interior = jnp.sum(m[ne + N_WORKERS:, :])
    lbl_row = lbl.reshape(1, ne)
    same = (lbl == lbl_row).astype(jnp.float32)
    gsum = jnp.dot(same, e, preferred_element_type=jnp.float32)
    gcnt = jnp.dot(same, cnt, preferred_element_type=jnp.float32)
    ir = lax.broadcasted_iota(jnp.int32, (ne, ne), 0)
    ic = lax.broadcasted_iota(jnp.int32, (ne, ne), 1)
    before = jnp.sum(same * (ic < ir).astype(jnp.float32), axis=1,
                     keepdims=True)
    first = (before == 0.0).astype(jnp.float32)
    gn2 = jnp.sum(gsum * gsum, axis=1, keepdims=True)
    contrib = jnp.sum(first * gn2 / jnp.maximum(gcnt, 1.0))
    out_ref[0, 0] = (sx - (interior + contrib)) / jnp.float32(N_ROWS * D)


_tc_combine = pl.pallas_call(
    _tc_combine_body,
    out_shape=jax.ShapeDtypeStruct((1, 1), jnp.float32),
    out_specs=pl.BlockSpec(memory_space=pltpu.SMEM),
)


def kernel(s_feature, s_labels):
    labels = s_labels.astype(jnp.int32)
    edge, meta = _sc_pass(s_feature, labels)
    out = _tc_combine(edge, meta)
    return out[0, 0]


# speedup vs baseline: 1.3063x; 1.3063x over previous
import functools

import jax
import jax.numpy as jnp
from jax import lax
from jax.experimental import pallas as pl
from jax.experimental.pallas import tpu as pltpu
from jax.experimental.pallas import tpu_sc as plsc

N_ROWS = 320000
D = 128
N_WORKERS = 32
ROWS_PER_W = N_ROWS // N_WORKERS
CHUNK = 200
N_CHUNKS = ROWS_PER_W // CHUNK
NV = D // 16


def _sc_body(feat_hbm, lbl_hbm, edge_out, meta_out,
             lbl_v, fbuf, ebuf, mbuf, sem0, sem1):
    c = lax.axis_index("c")
    s = lax.axis_index("s")
    wid = s * 2 + c
    row0 = wid * ROWS_PER_W

    pltpu.sync_copy(lbl_hbm.at[pl.ds(row0, ROWS_PER_W)],
                    lbl_v.at[pl.ds(0, ROWS_PER_W)])

    sems = (sem0, sem1)

    def start(ci, b):
        pltpu.async_copy(feat_hbm.at[pl.ds(row0 + ci * CHUNK, CHUNK)],
                         fbuf.at[b], sems[b])

    def wait(ci, b):
        pltpu.make_async_copy(feat_hbm.at[pl.ds(row0 + ci * CHUNK, CHUNK)],
                              fbuf.at[b], sems[b]).wait()

    start(0, 0)
    start(1, 1)

    zero_v = jnp.zeros((16,), jnp.float32)
    init = dict(
        prev=jnp.int32(-1),
        cnt=jnp.float32(0.0),
        nseg=jnp.int32(0),
        interior=zero_v,
        flbl=jnp.float32(0.0),
        fcnt=jnp.float32(0.0),
        acc=[zero_v] * NV,
        facc=[zero_v] * NV,
    )

    def row_step(b, gr0, r, st):
        gr = gr0 + r
        lbl = lbl_v[pl.ds(gr, 16)][0]
        boundary = jnp.logical_and(gr > 0, lbl != st["prev"])
        is_first = st["nseg"] == 0
        bf = jnp.logical_and(boundary, is_first)
        bi = jnp.logical_and(boundary, jnp.logical_not(is_first))

        acc = st["acc"]
        n2v = acc[0] * acc[0]
        for j in range(1, NV):
            n2v = n2v + acc[j] * acc[j]
        cntv = zero_v + st["cnt"]
        wv = jnp.where(bi, 1.0 / jnp.maximum(cntv, 1.0), zero_v)
        interior = st["interior"] + n2v * wv

        flbl = jnp.where(bf, st["prev"].astype(jnp.float32), st["flbl"])
        fcnt = jnp.where(bf, st["cnt"], st["fcnt"])
        facc = [jnp.where(bf, acc[j], st["facc"][j]) for j in range(NV)]
        nseg = jnp.where(boundary, st["nseg"] + 1, st["nseg"])
        cnt = jnp.where(boundary, jnp.float32(0.0), st["cnt"]) + 1.0

        new_acc = []
        for j in range(NV):
            v = fbuf[b, r, pl.ds(j * 16, 16)]
            a = jnp.where(boundary, zero_v, acc[j])
            new_acc.append(a + v)
        return dict(prev=lbl, cnt=cnt, nseg=nseg, interior=interior,
                    flbl=flbl, fcnt=fcnt, acc=new_acc, facc=facc)

    def row_pair(b, gr0, r, st):
        st = row_step(b, gr0, 2 * r, st)
        return row_step(b, gr0, 2 * r + 1, st)

    def chunk_pair(k, st):
        for b in range(2):
            ci = 2 * k + b
            wait(ci, b)
            st = lax.fori_loop(
                0, CHUNK // 2,
                functools.partial(row_pair, b, ci * CHUNK),
                st)

            @pl.when(ci + 2 < N_CHUNKS)
            def _():
                start(ci + 2, b)
        return st

    st = lax.fori_loop(0, N_CHUNKS // 2, chunk_pair, init)

    nseg = st["nseg"]
    last_lbl = st["prev"].astype(jnp.float32)
    flbl = jnp.where(nseg == 0, last_lbl, st["flbl"])
    fcnt = st["fcnt"]
    for j in range(NV):
        ebuf[0, pl.ds(j * 16, 16)] = st["facc"][j]
        ebuf[1, pl.ds(j * 16, 16)] = st["acc"][j]

    ii = lax.iota(jnp.int32, 16)
    zf = jnp.zeros((16,), jnp.float32)
    mfirst = jnp.where(ii == 0, flbl, zf) + jnp.where(ii == 1, fcnt, zf)
    mlast = (jnp.where(ii == 0, last_lbl, zf)
             + jnp.where(ii == 1, st["cnt"], zf))
    mbuf[0, pl.ds(0, 16)] = mfirst
    mbuf[1, pl.ds(0, 16)] = mlast
    mbuf[2, pl.ds(0, 16)] = zf
    mbuf[3, pl.ds(0, 16)] = st["interior"]

    pltpu.sync_copy(ebuf, edge_out.at[pl.ds(2 * wid, 2)])
    pltpu.sync_copy(mbuf.at[pl.ds(0, 2)], meta_out.at[pl.ds(2 * wid, 2)])
    pltpu.sync_copy(mbuf.at[pl.ds(2, 1)], meta_out.at[pl.ds(2 * N_WORKERS + wid, 1)])
    pltpu.sync_copy(mbuf.at[pl.ds(3, 1)],
                    meta_out.at[pl.ds(3 * N_WORKERS + wid, 1)])


_sc_pass = pl.kernel(
    _sc_body,
    out_type=[
        jax.ShapeDtypeStruct((2 * N_WORKERS, D), jnp.float32),
        jax.ShapeDtypeStruct((4 * N_WORKERS, 16), jnp.float32),
    ],
    mesh=plsc.VectorSubcoreMesh(core_axis_name="c", subcore_axis_name="s"),
    scratch_types=[
        pltpu.VMEM((ROWS_PER_W + 16,), jnp.int32),
        pltpu.VMEM((2, CHUNK, D), jnp.float32),
        pltpu.VMEM((2, D), jnp.float32),
        pltpu.VMEM((4, 16), jnp.float32),
        pltpu.SemaphoreType.DMA,
        pltpu.SemaphoreType.DMA,
    ],
)


SQ_BLK = 6400


def _tc_sumsq_body(x_ref, acc_ref):
    @pl.when(pl.program_id(0) == 0)
    def _():
        acc_ref[0, 0] = jnp.float32(0.0)
    x = x_ref[...]
    acc_ref[0, 0] += jnp.sum(x * x)


_tc_sumsq = pl.pallas_call(
    _tc_sumsq_body,
    grid=(N_ROWS // SQ_BLK,),
    in_specs=[pl.BlockSpec((SQ_BLK, D), lambda i: (i, 0))],
    out_shape=jax.ShapeDtypeStruct((1, 1), jnp.float32),
    out_specs=pl.BlockSpec(memory_space=pltpu.SMEM),
)


def _tc_combine_body(edge_ref, meta_ref, sq_ref, out_ref):
    ne = 2 * N_WORKERS
    e = edge_ref[...]
    m = meta_ref[...]
    lbl = m[:ne, 0:1]
    cnt = m[:ne, 1:2]
    sx = sq_ref[0, 0]
    interior = jnp.sum(m[ne + N_WORKERS:, :])
    lbl_row = lbl.reshape(1, ne)
    same = (lbl == lbl_row).astype(jnp.float32)
    gsum = jnp.dot(same, e, preferred_element_type=jnp.float32)
    gcnt = jnp.dot(same, cnt, preferred_element_type=jnp.float32)
    ir = lax.broadcasted_iota(jnp.int32, (ne, ne), 0)
    ic = lax.broadcasted_iota(jnp.int32, (ne, ne), 1)
    before = jnp.sum(same * (ic < ir).astype(jnp.float32), axis=1,
                     keepdims=True)
    first = (before == 0.0).astype(jnp.float32)
    gn2 = jnp.sum(gsum * gsum, axis=1, keepdims=True)
    contrib = jnp.sum(first * gn2 / jnp.maximum(gcnt, 1.0))
    out_ref[0, 0] = (sx - (interior + contrib)) / jnp.float32(N_ROWS * D)


_tc_combine = pl.pallas_call(
    _tc_combine_body,
    in_specs=[
        pl.BlockSpec(memory_space=pltpu.VMEM),
        pl.BlockSpec(memory_space=pltpu.VMEM),
        pl.BlockSpec(memory_space=pltpu.SMEM),
    ],
    out_shape=jax.ShapeDtypeStruct((1, 1), jnp.float32),
    out_specs=pl.BlockSpec(memory_space=pltpu.SMEM),
)


def kernel(s_feature, s_labels):
    labels = s_labels.astype(jnp.int32)
    sq = _tc_sumsq(s_feature)
    edge, meta = _sc_pass(s_feature, labels)
    out = _tc_combine(edge, meta, sq)
    return out[0, 0]
